# Initial kernel scaffold; baseline (speedup 1.0000x reference)
import jax
import jax.numpy as jnp
from jax.experimental import pallas as pl


def kernel(x, adj_t, W1, b1, W2, b2, W3, b3):
    raise NotImplementedError("write your pallas kernel here")


# speedup vs baseline: 11.6676x; 11.6676x over previous
import functools

import jax
import jax.numpy as jnp
from jax import lax
from jax.experimental import pallas as pl
from jax.experimental.pallas import tpu as pltpu
from jax.experimental.pallas import tpu_sc as plsc

N = 10000
E = 320000
D = 128

NC = 2
NS = 16
NW = NC * NS

EPW = E // NW
CHUNK = 80
NCHUNK = EPW // CHUNK
NPAD = 10240
ROWS_PW = NPAD // NS

_mesh = plsc.VectorSubcoreMesh(core_axis_name="c", subcore_axis_name="s",
                               num_cores=NC, num_subcores=NS)


def _sc_agg_body(h_hbm, src_hbm, dst_hbm, out_hbm, accum, src_v, dst_v,
                 buf0, buf1, sem0, sem1):
    c = lax.axis_index("c")
    s = lax.axis_index("s")
    w = c * NS + s

    pltpu.sync_copy(src_hbm.at[w], src_v)
    pltpu.sync_copy(dst_hbm.at[w], dst_v)

    zv = jnp.zeros((16,), jnp.float32)

    @pl.loop(0, CHUNK)
    def _(i):
        @pl.loop(0, D, step=16)
        def _(j):
            buf0[i, pl.ds(j, 16)] = zv

    @pl.loop(0, ROWS_PW // CHUNK)
    def _(k):
        pltpu.sync_copy(buf0, accum.at[pl.ds(s * ROWS_PW + k * CHUNK, CHUNK)])

    plsc.subcore_barrier()

    def gstart(ci, buf, sem):
        pltpu.async_copy(h_hbm.at[src_v.at[pl.ds(ci * CHUNK, CHUNK)]], buf,
                         sem)

    def gwait(ci, buf, sem):
        pltpu.make_async_copy(h_hbm.at[src_v.at[pl.ds(ci * CHUNK, CHUNK)]],
                              buf, sem).wait()

    def scat(ci, buf):
        pltpu.sync_copy(buf, accum.at[dst_v.at[ci]], add=True)

    gstart(0, buf0, sem0)

    @pl.loop(0, (NCHUNK - 1) // 2)
    def _(j):
        ci = 2 * j
        gstart(ci + 1, buf1, sem1)
        gwait(ci, buf0, sem0)
        scat(ci, buf0)
        gstart(ci + 2, buf0, sem0)
        gwait(ci + 1, buf1, sem1)
        scat(ci + 1, buf1)

    gwait(NCHUNK - 1, buf0, sem0)
    scat(NCHUNK - 1, buf0)

    plsc.subcore_barrier()

    pltpu.sync_copy(accum.at[pl.ds(s * ROWS_PW, ROWS_PW)],
                    out_hbm.at[c, pl.ds(s * ROWS_PW, ROWS_PW)])


def _sc_agg(h, adj_r):
    k = pl.kernel(
        _sc_agg_body,
        out_type=jax.ShapeDtypeStruct((NC, NPAD, D), jnp.float32),
        mesh=_mesh,
        scratch_types=[
            pltpu.VMEM_SHARED((NPAD, D), jnp.float32),
            pltpu.VMEM((EPW,), jnp.int32),
            pltpu.VMEM((NCHUNK, CHUNK), jnp.int32),
            pltpu.VMEM((CHUNK, D), jnp.float32),
            pltpu.VMEM((CHUNK, D), jnp.float32),
            pltpu.SemaphoreType.DMA,
            pltpu.SemaphoreType.DMA,
        ],
    )
    return k(h, adj_r[0].reshape(NW, EPW), adj_r[1])


BLK = 1000


def _mm_kernel(x_ref, w_ref, o_ref):
    o_ref[...] = jnp.dot(x_ref[...], w_ref[...],
                         preferred_element_type=jnp.float32)


def _mm(x, W):
    return pl.pallas_call(
        _mm_kernel,
        grid=(N // BLK,),
        in_specs=[pl.BlockSpec((BLK, D), lambda i: (i, 0)),
                  pl.BlockSpec((D, D), lambda i: (0, 0))],
        out_specs=pl.BlockSpec((BLK, D), lambda i: (i, 0)),
        out_shape=jax.ShapeDtypeStruct((N, D), jnp.float32),
    )(x, W)


def _mmc_kernel(p_ref, b_ref, w_ref, o_ref):
    h = p_ref[0] + p_ref[1] + b_ref[...]
    h = jnp.maximum(h, 0.0)
    o_ref[...] = jnp.dot(h, w_ref[...], preferred_element_type=jnp.float32)


def _mm_combine(p, b, W):
    return pl.pallas_call(
        _mmc_kernel,
        grid=(N // BLK,),
        in_specs=[pl.BlockSpec((NC, BLK, D), lambda i: (0, i, 0)),
                  pl.BlockSpec((1, D), lambda i: (0, 0)),
                  pl.BlockSpec((D, D), lambda i: (0, 0))],
        out_specs=pl.BlockSpec((BLK, D), lambda i: (i, 0)),
        out_shape=jax.ShapeDtypeStruct((N, D), jnp.float32),
    )(p, b.reshape(1, D), W)


def _final_kernel(p_ref, b_ref, o_ref):
    o_ref[...] = p_ref[0] + p_ref[1] + b_ref[...]


def _final(p, b):
    return pl.pallas_call(
        _final_kernel,
        grid=(N // BLK,),
        in_specs=[pl.BlockSpec((NC, BLK, D), lambda i: (0, i, 0)),
                  pl.BlockSpec((1, D), lambda i: (0, 0))],
        out_specs=pl.BlockSpec((BLK, D), lambda i: (i, 0)),
        out_shape=jax.ShapeDtypeStruct((N, D), jnp.float32),
    )(p, b.reshape(1, D))


def kernel(x, adj_t, W1, b1, W2, b2, W3, b3):
    adj_r = adj_t.reshape(2, NW, NCHUNK, CHUNK)
    h1 = _mm(x, W1)
    p1 = _sc_agg(h1, adj_r)
    h2 = _mm_combine(p1, b1, W2)
    p2 = _sc_agg(h2, adj_r)
    h3 = _mm_combine(p2, b2, W3)
    p3 = _sc_agg(h3, adj_r)
    return _final(p3, b3)
